# Initial kernel scaffold; baseline (speedup 1.0000x reference)
import jax
import jax.numpy as jnp
from jax.experimental import pallas as pl


def kernel(opacities, values, depths, origins, dirs):
    raise NotImplementedError("write your pallas kernel here")


# speedup vs baseline: 1.3527x; 1.3527x over previous
import functools

import jax
import jax.numpy as jnp
import numpy as np
from jax.experimental import pallas as pl

_PTS = 64
_IMP = 64
_EPS = 1e-5

_T_INC = np.triu(np.ones((_PTS, _PTS), np.float32))
_T_EXC = np.triu(np.ones((_PTS, _PTS), np.float32), k=1)
_U = np.linspace(0.0, 1.0, _IMP + 1, dtype=np.float32)


def _body(op_ref, d_ref, v0_ref, v1_ref, v2_ref, o_ref, dir_ref,
          tin_ref, tex_ref, ox_ref, oy_ref, oz_ref):
    op = op_ref[...]
    d = d_ref[...]
    tin = tin_ref[...]
    tex = tex_ref[...]

    l = jnp.log(1.0 - op)
    absorption = jnp.exp(
        jax.lax.dot(l, tex, precision=jax.lax.Precision.HIGHEST))
    w = op * absorption

    accs = []
    for vref in (v0_ref, v1_ref, v2_ref):
        accs.append(jnp.sum(w * vref[...], axis=-1, keepdims=True))
    acc_o = jnp.clip(jnp.sum(w, axis=-1, keepdims=True), 0.0, 1.0)

    wp = w + _EPS
    cdf = jax.lax.dot(wp, tin, precision=jax.lax.Precision.HIGHEST)
    ctot = cdf[:, _PTS - 1:_PTS]
    c_first = cdf[:, 0:1]
    d_first = d[:, 0:1]
    d_last = d[:, _PTS - 1:_PTS]

    inf = jnp.float32(jnp.inf)
    fs = []
    for j in range(_IMP + 1):
        uj = ctot * float(_U[j])
        mask = cdf <= uj
        c0 = jnp.max(jnp.where(mask, cdf, -inf), axis=-1, keepdims=True)
        c1 = jnp.min(jnp.where(mask, inf, cdf), axis=-1, keepdims=True)
        d0 = jnp.max(jnp.where(mask, d, -inf), axis=-1, keepdims=True)
        d1 = jnp.min(jnp.where(mask, inf, d), axis=-1, keepdims=True)
        t = (uj - c0) / (c1 - c0)
        f = d0 + t * (d1 - d0)
        f = jnp.where(uj < c_first, d_first, f)
        f = jnp.where(uj >= ctot, d_last, f)
        fs.append(f)
    fgrid = jnp.concatenate(fs, axis=1)
    mid = 0.5 * (fgrid[:, :-1] + fgrid[:, 1:])

    o = o_ref[...]
    dr = dir_ref[...]
    for c, out_ref, acc in ((0, ox_ref, accs[0]), (1, oy_ref, accs[1]),
                            (2, oz_ref, accs[2])):
        coords = o[:, c:c + 1] + mid * dr[:, c:c + 1]
        out_ref[...] = jnp.concatenate([acc, acc_o, coords], axis=1)


def kernel(opacities, values, depths, origins, dirs):
    n = opacities.shape[0]
    r = 256 if n % 256 == 0 else n
    grid = n // r
    v0 = values[:, :, 0]
    v1 = values[:, :, 1]
    v2 = values[:, :, 2]

    row = lambda i: (i, 0)
    in_specs = [
        pl.BlockSpec((r, _PTS), row),
        pl.BlockSpec((r, _PTS), row),
        pl.BlockSpec((r, _PTS), row),
        pl.BlockSpec((r, _PTS), row),
        pl.BlockSpec((r, _PTS), row),
        pl.BlockSpec((r, 3), row),
        pl.BlockSpec((r, 3), row),
        pl.BlockSpec((_PTS, _PTS), lambda i: (0, 0)),
        pl.BlockSpec((_PTS, _PTS), lambda i: (0, 0)),
    ]
    out_specs = [pl.BlockSpec((r, _IMP + 2), row)] * 3
    out_shape = [jax.ShapeDtypeStruct((n, _IMP + 2), jnp.float32)] * 3

    ox, oy, oz = pl.pallas_call(
        _body,
        grid=(grid,),
        in_specs=in_specs,
        out_specs=out_specs,
        out_shape=out_shape,
    )(opacities, depths, v0, v1, v2, origins, dirs,
      jnp.asarray(_T_INC), jnp.asarray(_T_EXC))
    return jnp.stack([ox, oy, oz], axis=-1)
